# Initial kernel scaffold; baseline (speedup 1.0000x reference)
import jax
import jax.numpy as jnp
from jax.experimental import pallas as pl


def kernel(unique_emb, history_0, history_1, history_2, label_0, label_1, label_2, W_0, W_1, W_2, b_0, b_1, b_2):
    raise NotImplementedError("write your pallas kernel here")


# speedup vs baseline: 3.5351x; 3.5351x over previous
import functools

import jax
import jax.numpy as jnp
from jax import lax
from jax.experimental import pallas as pl
from jax.experimental.pallas import tpu as pltpu
from jax.experimental.pallas import tpu_sc as plsc

VOCAB = 1000000
D = 32
B = 4096
HL = 200
C = 6
EPS = 1e-9

NC, NS = 2, 16
NW = NC * NS
BAGS = 3 * B
BPW = BAGS // NW
CH = 16

def _emb_bag_body(hist, table, out, idx_v, rows_v, acc_v, sem):
    wid = lax.axis_index("s") * NC + lax.axis_index("c")
    base = wid * BPW
    zero = jnp.zeros((16,), jnp.float32)

    def chunk_body(ci, _):
        row0 = base + ci * CH
        pltpu.sync_copy(hist.at[pl.ds(row0, CH)], idx_v)

        def bag_body(j, _):
            cp0 = pltpu.async_copy(
                table.at[idx_v.at[j, pl.ds(0, 128)]], rows_v.at[pl.ds(0, 128)], sem)
            cp1 = pltpu.async_copy(
                table.at[idx_v.at[j, pl.ds(128, HL - 128)]],
                rows_v.at[pl.ds(128, HL - 128)], sem)
            cp0.wait()
            cp1.wait()

            def accum(r, accs):
                a0, a1 = accs
                return (a0 + rows_v[r, pl.ds(0, 16)],
                        a1 + rows_v[r, pl.ds(16, 16)])

            a0, a1 = lax.fori_loop(0, HL, accum, (zero, zero), unroll=8)
            bi = ci * CH + j
            acc_v[bi, pl.ds(0, 16)] = a0
            acc_v[bi, pl.ds(16, 16)] = a1
            return 0

        lax.fori_loop(0, CH, bag_body, 0)
        return 0

    lax.fori_loop(0, BPW // CH, chunk_body, 0)
    pltpu.sync_copy(acc_v, out.at[pl.ds(base, BPW)])


@functools.cache
def _emb_bag():
    mesh = plsc.VectorSubcoreMesh(
        core_axis_name="c", subcore_axis_name="s", num_cores=NC, num_subcores=NS)
    return pl.kernel(
        _emb_bag_body,
        out_type=jax.ShapeDtypeStruct((BAGS, D), jnp.float32),
        mesh=mesh,
        scratch_types=[
            pltpu.VMEM((CH, HL), jnp.int32),
            pltpu.VMEM((HL, D), jnp.float32),
            pltpu.VMEM((BPW, D), jnp.float32),
            pltpu.SemaphoreType.DMA,
        ],
        compiler_params=pltpu.CompilerParams(use_tc_tiling_on_sc=False),
    )


def _tail_body(pe0, pe1, pe2, lb0, lb1, lb2, w0, w1, w2, bb0, bb1, bb2, out):
    loss_sum = jnp.float32(0.0)
    pos_tp = jnp.float32(0.0); pos_fp = jnp.float32(0.0); pos_fn = jnp.float32(0.0)
    neg_tp = jnp.float32(0.0); neg_fp = jnp.float32(0.0); neg_fn = jnp.float32(0.0)
    correct = jnp.float32(0.0)

    for pe, lb, w, bb in ((pe0, lb0, w0, bb0), (pe1, lb1, w1, bb1),
                          (pe2, lb2, w2, bb2)):
        x = pe[...]
        sq = jnp.sum(x * x, axis=1, keepdims=True)
        nx = x * lax.rsqrt(jnp.maximum(sq, 1e-12))
        logits = jnp.dot(nx, w[...], preferred_element_type=jnp.float32) + bb[...]
        pred = jnp.clip(jax.nn.sigmoid(logits), EPS, 1.0 - EPS)
        lbl = lb[...]
        loss = -lbl * jnp.log(pred) - (1.0 - lbl) * jnp.log(1.0 - pred)
        loss_sum = loss_sum + jnp.sum(loss) / jnp.float32(B)

        predp = pred > 0.5
        blp = lbl == 1.0
        f = lambda m: jnp.sum(jnp.where(m, 1.0, 0.0))
        correct = correct + f(predp == blp)
        pos_tp = pos_tp + f(jnp.logical_and(blp, predp))
        pos_fp = pos_fp + f(jnp.logical_and(jnp.logical_not(blp), predp))
        pos_fn = pos_fn + f(jnp.logical_and(blp, jnp.logical_not(predp)))

        predn = pred < 0.5
        bln = lbl == 0.0
        neg_tp = neg_tp + f(jnp.logical_and(bln, predn))
        neg_fp = neg_fp + f(jnp.logical_and(jnp.logical_not(bln), predn))
        neg_fn = neg_fn + f(jnp.logical_and(bln, jnp.logical_not(predn)))

    accuracy = correct / jnp.float32(B * 18)
    pos_recall = pos_tp / jnp.maximum(EPS, pos_tp + pos_fn)
    pos_precision = pos_tp / jnp.maximum(EPS, pos_tp + pos_fp)
    pos_f1 = 2 * pos_recall * pos_precision / jnp.maximum(EPS, pos_recall + pos_precision)
    neg_recall = neg_tp / jnp.maximum(EPS, neg_tp + neg_fn)
    neg_precision = neg_tp / jnp.maximum(EPS, neg_tp + neg_fp)
    neg_f1 = 2 * neg_recall * neg_precision / jnp.maximum(EPS, neg_recall + neg_precision)

    out[0] = loss_sum
    out[1] = (pos_f1 + neg_f1) / 2.0
    out[2] = accuracy


_tail = pl.pallas_call(
    _tail_body,
    out_shape=jax.ShapeDtypeStruct((3,), jnp.float32),
    out_specs=pl.BlockSpec(memory_space=pltpu.SMEM),
)


def kernel(unique_emb, history_0, history_1, history_2,
           label_0, label_1, label_2,
           W_0, W_1, W_2, b_0, b_1, b_2):
    hist = jnp.concatenate([history_0, history_1, history_2], axis=0)
    pooled = _emb_bag()(hist, unique_emb)
    out = _tail(pooled[:B], pooled[B:2 * B], pooled[2 * B:],
                label_0, label_1, label_2,
                W_0, W_1, W_2,
                b_0.reshape(1, C), b_1.reshape(1, C), b_2.reshape(1, C))
    return (out[0], out[1], out[2])


# speedup vs baseline: 5.1902x; 1.4682x over previous
import functools

import jax
import jax.numpy as jnp
from jax import lax
from jax.experimental import pallas as pl
from jax.experimental.pallas import tpu as pltpu
from jax.experimental.pallas import tpu_sc as plsc

VOCAB = 1000000
D = 32
B = 4096
HL = 200
C = 6
EPS = 1e-9

NC, NS = 2, 16
NW = NC * NS
BAGS = 3 * B
BPW = BAGS // NW
CH = 16

NBUF = 4


def _emb_bag_body(hist, table, out, idx_v, rows_v, acc_v, sems):
    wid = lax.axis_index("s") * NC + lax.axis_index("c")
    base = wid * BPW
    zero = jnp.zeros((16,), jnp.float32)

    pltpu.sync_copy(hist.at[pl.ds(base, BPW)], idx_v)

    def fire(j, p):
        pltpu.async_copy(table.at[idx_v.at[j, pl.ds(0, 128)]],
                         rows_v.at[p, pl.ds(0, 128)], sems[p])
        pltpu.async_copy(table.at[idx_v.at[j, pl.ds(128, HL - 128)]],
                         rows_v.at[p, pl.ds(128, HL - 128)], sems[p])

    def wait(p):
        pltpu.make_async_copy(table.at[idx_v.at[0]], rows_v.at[p], sems[p]).wait()

    for p in range(NBUF):
        fire(p, p)

    NT = BPW // NBUF

    def it(t, _):
        for p in range(NBUF):
            wait(p)

            def accum(r, accs, p=p):
                a0, a1 = accs
                return (a0 + rows_v[p, r, pl.ds(0, 16)],
                        a1 + rows_v[p, r, pl.ds(16, 16)])

            a0, a1 = lax.fori_loop(0, HL, accum, (zero, zero), unroll=8)
            bi = t * NBUF + p
            acc_v[bi, pl.ds(0, 16)] = a0
            acc_v[bi, pl.ds(16, 16)] = a1

            @pl.when(t < NT - 1)
            def _(bi=bi, p=p):
                fire(bi + NBUF, p)
        return 0

    lax.fori_loop(0, NT, it, 0)
    pltpu.sync_copy(acc_v, out.at[pl.ds(base, BPW)])


@functools.cache
def _emb_bag():
    mesh = plsc.VectorSubcoreMesh(
        core_axis_name="c", subcore_axis_name="s", num_cores=NC, num_subcores=NS)
    return pl.kernel(
        _emb_bag_body,
        out_type=jax.ShapeDtypeStruct((BAGS, D), jnp.float32),
        mesh=mesh,
        scratch_types=[
            pltpu.VMEM((BPW, HL), jnp.int32),
            pltpu.VMEM((NBUF, HL, D), jnp.float32),
            pltpu.VMEM((BPW, D), jnp.float32),
            [pltpu.SemaphoreType.DMA] * NBUF,
        ],
        compiler_params=pltpu.CompilerParams(use_tc_tiling_on_sc=False),
    )


def _tail_body(pe0, pe1, pe2, lb0, lb1, lb2, w0, w1, w2, bb0, bb1, bb2, out):
    loss_sum = jnp.float32(0.0)
    pos_tp = jnp.float32(0.0); pos_fp = jnp.float32(0.0); pos_fn = jnp.float32(0.0)
    neg_tp = jnp.float32(0.0); neg_fp = jnp.float32(0.0); neg_fn = jnp.float32(0.0)
    correct = jnp.float32(0.0)

    for pe, lb, w, bb in ((pe0, lb0, w0, bb0), (pe1, lb1, w1, bb1),
                          (pe2, lb2, w2, bb2)):
        x = pe[...]
        sq = jnp.sum(x * x, axis=1, keepdims=True)
        nx = x * lax.rsqrt(jnp.maximum(sq, 1e-12))
        logits = jnp.dot(nx, w[...], preferred_element_type=jnp.float32) + bb[...]
        pred = jnp.clip(jax.nn.sigmoid(logits), EPS, 1.0 - EPS)
        lbl = lb[...]
        loss = -lbl * jnp.log(pred) - (1.0 - lbl) * jnp.log(1.0 - pred)
        loss_sum = loss_sum + jnp.sum(loss) / jnp.float32(B)

        predp = pred > 0.5
        blp = lbl == 1.0
        f = lambda m: jnp.sum(jnp.where(m, 1.0, 0.0))
        correct = correct + f(predp == blp)
        pos_tp = pos_tp + f(jnp.logical_and(blp, predp))
        pos_fp = pos_fp + f(jnp.logical_and(jnp.logical_not(blp), predp))
        pos_fn = pos_fn + f(jnp.logical_and(blp, jnp.logical_not(predp)))

        predn = pred < 0.5
        bln = lbl == 0.0
        neg_tp = neg_tp + f(jnp.logical_and(bln, predn))
        neg_fp = neg_fp + f(jnp.logical_and(jnp.logical_not(bln), predn))
        neg_fn = neg_fn + f(jnp.logical_and(bln, jnp.logical_not(predn)))

    accuracy = correct / jnp.float32(B * 18)
    pos_recall = pos_tp / jnp.maximum(EPS, pos_tp + pos_fn)
    pos_precision = pos_tp / jnp.maximum(EPS, pos_tp + pos_fp)
    pos_f1 = 2 * pos_recall * pos_precision / jnp.maximum(EPS, pos_recall + pos_precision)
    neg_recall = neg_tp / jnp.maximum(EPS, neg_tp + neg_fn)
    neg_precision = neg_tp / jnp.maximum(EPS, neg_tp + neg_fp)
    neg_f1 = 2 * neg_recall * neg_precision / jnp.maximum(EPS, neg_recall + neg_precision)

    out[0] = loss_sum
    out[1] = (pos_f1 + neg_f1) / 2.0
    out[2] = accuracy


_tail = pl.pallas_call(
    _tail_body,
    out_shape=jax.ShapeDtypeStruct((3,), jnp.float32),
    out_specs=pl.BlockSpec(memory_space=pltpu.SMEM),
)


def kernel(unique_emb, history_0, history_1, history_2,
           label_0, label_1, label_2,
           W_0, W_1, W_2, b_0, b_1, b_2):
    hist = jnp.concatenate([history_0, history_1, history_2], axis=0)
    pooled = _emb_bag()(hist, unique_emb)
    out = _tail(pooled[:B], pooled[B:2 * B], pooled[2 * B:],
                label_0, label_1, label_2,
                W_0, W_1, W_2,
                b_0.reshape(1, C), b_1.reshape(1, C), b_2.reshape(1, C))
    return (out[0], out[1], out[2])


# speedup vs baseline: 5.2645x; 1.0143x over previous
import functools

import jax
import jax.numpy as jnp
from jax import lax
from jax.experimental import pallas as pl
from jax.experimental.pallas import tpu as pltpu
from jax.experimental.pallas import tpu_sc as plsc

VOCAB = 1000000
D = 32
B = 4096
HL = 200
C = 6
EPS = 1e-9

NC, NS = 2, 16
NW = NC * NS
BAGS = 3 * B
BPW = BAGS // NW
CH = 16

NBUF = 4
BPT = B // NW


def _emb_bag_body(hist0, hist1, hist2, table, out0, out1, out2,
                  idx_v, rows_v, acc_v, sems):
    wid = lax.axis_index("s") * NC + lax.axis_index("c")
    zero = jnp.zeros((16,), jnp.float32)

    for t, hist in enumerate((hist0, hist1, hist2)):
        pltpu.sync_copy(hist.at[pl.ds(wid * BPT, BPT)],
                        idx_v.at[pl.ds(t * BPT, BPT)])

    def fire(j, p):
        pltpu.async_copy(table.at[idx_v.at[j, pl.ds(0, 128)]],
                         rows_v.at[p, pl.ds(0, 128)], sems[p])
        pltpu.async_copy(table.at[idx_v.at[j, pl.ds(128, HL - 128)]],
                         rows_v.at[p, pl.ds(128, HL - 128)], sems[p])

    def wait(p):
        pltpu.make_async_copy(table.at[idx_v.at[0]], rows_v.at[p], sems[p]).wait()

    for p in range(NBUF):
        fire(p, p)

    NT = BPW // NBUF

    def it(t, _):
        for p in range(NBUF):
            wait(p)

            def accum(r, accs, p=p):
                a0, a1 = accs
                return (a0 + rows_v[p, r, pl.ds(0, 16)],
                        a1 + rows_v[p, r, pl.ds(16, 16)])

            a0, a1 = lax.fori_loop(0, HL, accum, (zero, zero), unroll=8)
            bi = t * NBUF + p
            acc_v[bi, pl.ds(0, 16)] = a0
            acc_v[bi, pl.ds(16, 16)] = a1

            @pl.when(t < NT - 1)
            def _(bi=bi, p=p):
                fire(bi + NBUF, p)
        return 0

    lax.fori_loop(0, NT, it, 0)
    for t, out in enumerate((out0, out1, out2)):
        pltpu.sync_copy(acc_v.at[pl.ds(t * BPT, BPT)],
                        out.at[pl.ds(wid * BPT, BPT)])


@functools.cache
def _emb_bag():
    mesh = plsc.VectorSubcoreMesh(
        core_axis_name="c", subcore_axis_name="s", num_cores=NC, num_subcores=NS)
    pooled = jax.ShapeDtypeStruct((B, D), jnp.float32)
    return pl.kernel(
        _emb_bag_body,
        out_type=(pooled, pooled, pooled),
        mesh=mesh,
        scratch_types=[
            pltpu.VMEM((BPW, HL), jnp.int32),
            pltpu.VMEM((NBUF, HL, D), jnp.float32),
            pltpu.VMEM((BPW, D), jnp.float32),
            [pltpu.SemaphoreType.DMA] * NBUF,
        ],
        compiler_params=pltpu.CompilerParams(use_tc_tiling_on_sc=False),
    )


def _tail_body(pe0, pe1, pe2, lb0, lb1, lb2, w0, w1, w2, bb0, bb1, bb2, out):
    loss_sum = jnp.float32(0.0)
    pos_tp = jnp.float32(0.0); pos_fp = jnp.float32(0.0); pos_fn = jnp.float32(0.0)
    neg_tp = jnp.float32(0.0); neg_fp = jnp.float32(0.0); neg_fn = jnp.float32(0.0)
    correct = jnp.float32(0.0)

    for pe, lb, w, bb in ((pe0, lb0, w0, bb0), (pe1, lb1, w1, bb1),
                          (pe2, lb2, w2, bb2)):
        x = pe[...]
        sq = jnp.sum(x * x, axis=1, keepdims=True)
        nx = x * lax.rsqrt(jnp.maximum(sq, 1e-12))
        logits = jnp.dot(nx, w[...], preferred_element_type=jnp.float32) + bb[...]
        pred = jnp.clip(jax.nn.sigmoid(logits), EPS, 1.0 - EPS)
        lbl = lb[...]
        loss = -lbl * jnp.log(pred) - (1.0 - lbl) * jnp.log(1.0 - pred)
        loss_sum = loss_sum + jnp.sum(loss) / jnp.float32(B)

        predp = pred > 0.5
        blp = lbl == 1.0
        f = lambda m: jnp.sum(jnp.where(m, 1.0, 0.0))
        correct = correct + f(predp == blp)
        pos_tp = pos_tp + f(jnp.logical_and(blp, predp))
        pos_fp = pos_fp + f(jnp.logical_and(jnp.logical_not(blp), predp))
        pos_fn = pos_fn + f(jnp.logical_and(blp, jnp.logical_not(predp)))

        predn = pred < 0.5
        bln = lbl == 0.0
        neg_tp = neg_tp + f(jnp.logical_and(bln, predn))
        neg_fp = neg_fp + f(jnp.logical_and(jnp.logical_not(bln), predn))
        neg_fn = neg_fn + f(jnp.logical_and(bln, jnp.logical_not(predn)))

    accuracy = correct / jnp.float32(B * 18)
    pos_recall = pos_tp / jnp.maximum(EPS, pos_tp + pos_fn)
    pos_precision = pos_tp / jnp.maximum(EPS, pos_tp + pos_fp)
    pos_f1 = 2 * pos_recall * pos_precision / jnp.maximum(EPS, pos_recall + pos_precision)
    neg_recall = neg_tp / jnp.maximum(EPS, neg_tp + neg_fn)
    neg_precision = neg_tp / jnp.maximum(EPS, neg_tp + neg_fp)
    neg_f1 = 2 * neg_recall * neg_precision / jnp.maximum(EPS, neg_recall + neg_precision)

    out[0] = loss_sum
    out[1] = (pos_f1 + neg_f1) / 2.0
    out[2] = accuracy


_tail = pl.pallas_call(
    _tail_body,
    out_shape=jax.ShapeDtypeStruct((3,), jnp.float32),
    out_specs=pl.BlockSpec(memory_space=pltpu.SMEM),
)


def kernel(unique_emb, history_0, history_1, history_2,
           label_0, label_1, label_2,
           W_0, W_1, W_2, b_0, b_1, b_2):
    p0, p1, p2 = _emb_bag()(history_0, history_1, history_2, unique_emb)
    out = _tail(p0, p1, p2,
                label_0, label_1, label_2,
                W_0, W_1, W_2,
                b_0.reshape(1, C), b_1.reshape(1, C), b_2.reshape(1, C))
    return (out[0], out[1], out[2])
